# baseline (speedup 1.0000x reference)
import jax
import jax.numpy as jnp
from jax import lax
from jax.experimental import pallas as pl
from jax.experimental.pallas import tpu as pltpu
from jax.experimental.pallas import tpu_sc as plsc

S = 2048
D = 1024
DH = 4096
E = 8
K = 2

C = 128
NB = 40
NPAD = NB * C

DHC1 = 1024
J1 = DH // DHC1
DC2 = 512
J2 = D // DC2

NW = 32
RPW = NPAD // NW
GCH = 4
RPC = RPW // GCH
TPW = S // NW
CCH = 2
TPC = TPW // CCH

_SQRT1_2 = 0.7071067811865476


def _gelu(h):
    return 0.5 * h * (1.0 + lax.erf(h * _SQRT1_2))


def _sc_mesh():
    return plsc.VectorSubcoreMesh(core_axis_name="c", subcore_axis_name="s",
                                  num_cores=2, num_subcores=16)


def _gating_kernel(x_ref, gw_ref, gb_ref,
                   gpos0_ref, gpos1_ref, w0_ref, w1_ref, bend_ref):
    logits = lax.dot_general(
        x_ref[...], gw_ref[...], (((1,), (1,)), ((), ())),
        preferred_element_type=jnp.float32) + gb_ref[...]
    lane = lax.broadcasted_iota(jnp.int32, (S, E), 1)

    m0 = jnp.max(logits, axis=1, keepdims=True)
    i0 = jnp.min(jnp.where(logits == m0, lane, E), axis=1, keepdims=True)
    l1 = jnp.where(lane == i0, -jnp.inf, logits)
    m1 = jnp.max(l1, axis=1, keepdims=True)
    i1 = jnp.min(jnp.where(l1 == m1, lane, E), axis=1, keepdims=True)
    e1 = jnp.exp(m1 - m0)
    w0 = 1.0 / (1.0 + e1)
    w1 = 1.0 - w0

    cnt = (lane == i0).astype(jnp.int32) + (lane == i1).astype(jnp.int32)
    incl = cnt
    k = 1
    while k < S:
        shifted = jnp.concatenate(
            [jnp.zeros((k, E), jnp.int32), incl[:-k]], axis=0)
        incl = incl + shifted
        k *= 2
    excl = incl - cnt

    counts = incl[S - 1:S, :]
    nb = (counts + (C - 1)) // C
    bincl = nb
    k = 1
    while k < E:
        shifted = jnp.concatenate(
            [jnp.zeros((1, k), jnp.int32), bincl[:, :-k]], axis=1)
        bincl = bincl + shifted
        k *= 2
    bstart = bincl - nb
    pstart = C * bstart

    pstart_b = jnp.broadcast_to(pstart, (S, E))
    pos0 = jnp.sum(jnp.where(lane == i0, excl + pstart_b, 0),
                   axis=1, keepdims=True)
    pos1 = jnp.sum(jnp.where(lane == i1, excl + pstart_b, 0),
                   axis=1, keepdims=True)

    gpos0_ref[...] = pos0
    gpos1_ref[...] = pos1
    w0_ref[...] = w0
    w1_ref[...] = w1
    bend_ref[...] = bincl


def _run_gating(x_flat, gate_W, gb):
    return pl.pallas_call(
        _gating_kernel,
        out_shape=(
            jax.ShapeDtypeStruct((S, 1), jnp.int32),
            jax.ShapeDtypeStruct((S, 1), jnp.int32),
            jax.ShapeDtypeStruct((S, 1), jnp.float32),
            jax.ShapeDtypeStruct((S, 1), jnp.float32),
            jax.ShapeDtypeStruct((1, E), jnp.int32),
        ),
    )(x_flat, gate_W, gb)


def _sc_gather_body(x_hbm, idx_hbm, out_hbm, idx_v, rows_v, gsem, wsem):
    wid = lax.axis_index("s") * 2 + lax.axis_index("c")
    base = wid * RPW
    pltpu.sync_copy(idx_hbm.at[wid], idx_v)
    gets = [None] * GCH
    puts = [None] * GCH
    gets[0] = pltpu.async_copy(x_hbm.at[idx_v.at[0]], rows_v.at[0], gsem)
    for c in range(GCH):
        gets[c].wait()
        if c > 0:
            puts[c - 1].wait()
        if c + 1 < GCH:
            gets[c + 1] = pltpu.async_copy(
                x_hbm.at[idx_v.at[c + 1]], rows_v.at[(c + 1) % 2], gsem)
        puts[c] = pltpu.async_copy(
            rows_v.at[c % 2], out_hbm.at[pl.ds(base + c * RPC, RPC)], wsem)
    puts[GCH - 1].wait()


def _run_sc_gather(x_flat, row_token):
    f = pl.kernel(
        _sc_gather_body,
        out_type=jax.ShapeDtypeStruct((NPAD, D), jnp.float32),
        mesh=_sc_mesh(),
        scratch_types=[
            pltpu.VMEM((GCH, RPC), jnp.int32),
            pltpu.VMEM((2, RPC, D), jnp.float32),
            pltpu.SemaphoreType.DMA,
            pltpu.SemaphoreType.DMA,
        ],
    )
    return f(x_flat, row_token.reshape(NW, GCH, RPC))


def _ffn1_kernel(be_ref, nbt_ref, xs_ref, w1_ref, b1_ref, wr_ref, h_ref):
    b = pl.program_id(1)

    @pl.when(b < nbt_ref[0])
    def _():
        h = lax.dot_general(xs_ref[...].astype(jnp.bfloat16),
                            w1_ref[0].astype(jnp.bfloat16),
                            (((1,), (0,)), ((), ())),
                            preferred_element_type=jnp.float32)
        h = _gelu(h + b1_ref[0])
        h_ref[...] = (h * wr_ref[...]).astype(jnp.bfloat16)


def _run_ffn1(xs, W1, b1r, w_row, be, nbt):
    grid_spec = pltpu.PrefetchScalarGridSpec(
        num_scalar_prefetch=2,
        grid=(J1, NB),
        in_specs=[
            pl.BlockSpec((C, D), lambda j, b, be, nbt: (b, 0)),
            pl.BlockSpec((1, D, DHC1), lambda j, b, be, nbt: (be[b], 0, j)),
            pl.BlockSpec((1, 1, DHC1), lambda j, b, be, nbt: (be[b], 0, j)),
            pl.BlockSpec((C, 1), lambda j, b, be, nbt: (b, 0)),
        ],
        out_specs=pl.BlockSpec((C, DHC1), lambda j, b, be, nbt: (b, j)),
    )
    return pl.pallas_call(
        _ffn1_kernel,
        grid_spec=grid_spec,
        out_shape=jax.ShapeDtypeStruct((NPAD, DH), jnp.bfloat16),
    )(be, nbt, xs, W1, b1r, w_row)


def _ffn2_kernel(be_ref, nbt_ref, h_ref, w2_ref, b2_ref, wr_ref, ys_ref):
    b = pl.program_id(1)

    @pl.when(b < nbt_ref[0])
    def _():
        yb = lax.dot_general(h_ref[...], w2_ref[0].astype(jnp.bfloat16),
                             (((1,), (0,)), ((), ())),
                             preferred_element_type=jnp.float32)
        ys_ref[...] = yb + wr_ref[...] * b2_ref[0]


def _run_ffn2(h, W2, b2r, w_row, be, nbt):
    grid_spec = pltpu.PrefetchScalarGridSpec(
        num_scalar_prefetch=2,
        grid=(J2, NB),
        in_specs=[
            pl.BlockSpec((C, DH), lambda j, b, be, nbt: (b, 0)),
            pl.BlockSpec((1, DH, DC2), lambda j, b, be, nbt: (be[b], 0, j)),
            pl.BlockSpec((1, 1, DC2), lambda j, b, be, nbt: (be[b], 0, j)),
            pl.BlockSpec((C, 1), lambda j, b, be, nbt: (b, 0)),
        ],
        out_specs=pl.BlockSpec((C, DC2), lambda j, b, be, nbt: (b, j)),
    )
    return pl.pallas_call(
        _ffn2_kernel,
        grid_spec=grid_spec,
        out_shape=jax.ShapeDtypeStruct((NPAD, D), jnp.float32),
    )(be, nbt, h, W2, b2r, w_row)


def _sc_combine_body(ys_hbm, g0_hbm, g1_hbm, out_hbm,
                     g0_v, g1_v, r0_v, r1_v, sem0, sem1):
    wid = lax.axis_index("s") * 2 + lax.axis_index("c")
    base = wid * TPW
    pltpu.sync_copy(g0_hbm.at[wid], g0_v)
    pltpu.sync_copy(g1_hbm.at[wid], g1_v)
    for c in range(CCH):
        cp0 = pltpu.async_copy(ys_hbm.at[g0_v.at[c]], r0_v, sem0)
        cp1 = pltpu.async_copy(ys_hbm.at[g1_v.at[c]], r1_v, sem1)
        cp0.wait()
        cp1.wait()

        def tok(t, carry):
            for v in range(D // 16):
                sl = pl.ds(v * 16, 16)
                r0_v[t, sl] = r0_v[t, sl] + r1_v[t, sl]
            return carry

        lax.fori_loop(0, TPC, tok, 0)
        pltpu.sync_copy(r0_v, out_hbm.at[pl.ds(base + c * TPC, TPC)])


def _run_sc_combine(ys, g0, g1):
    f = pl.kernel(
        _sc_combine_body,
        out_type=jax.ShapeDtypeStruct((S, D), jnp.float32),
        mesh=_sc_mesh(),
        scratch_types=[
            pltpu.VMEM((CCH, TPC), jnp.int32),
            pltpu.VMEM((CCH, TPC), jnp.int32),
            pltpu.VMEM((TPC, D), jnp.float32),
            pltpu.VMEM((TPC, D), jnp.float32),
            pltpu.SemaphoreType.DMA,
            pltpu.SemaphoreType.DMA,
        ],
    )
    return f(ys, g0.reshape(NW, CCH, TPC), g1.reshape(NW, CCH, TPC))


def kernel(x, gate_W, gate_b, W1, b1, W2, b2):
    B = x.shape[0]
    x_flat = x.reshape(S, D)
    gb = gate_b.reshape(1, E)
    b1r = b1.reshape(E, 1, DH)
    b2r = b2.reshape(E, 1, D)

    gpos0, gpos1, w0, w1, bend = _run_gating(x_flat, gate_W, gb)
    gp0 = gpos0.reshape(S)
    gp1 = gpos1.reshape(S)
    bend_f = bend.reshape(E)

    tok_ids = jnp.arange(S, dtype=jnp.int32)
    cat_idx = jnp.concatenate([gp0, gp1])
    row_token = (jnp.zeros((NPAD,), jnp.int32)
                 .at[cat_idx].set(jnp.concatenate([tok_ids, tok_ids])))
    w_row = (jnp.zeros((NPAD,), jnp.float32)
             .at[cat_idx].set(jnp.concatenate([w0.reshape(S),
                                               w1.reshape(S)]))
             ).reshape(NPAD, 1)
    be = jnp.minimum(
        jnp.sum((jnp.arange(NB, dtype=jnp.int32)[:, None]
                 >= bend_f[None, :]).astype(jnp.int32), axis=1),
        E - 1)
    nbt = bend_f[E - 1:E]

    xs = _run_sc_gather(x_flat, row_token)
    h = _run_ffn1(xs, W1, b1r, w_row, be, nbt)
    ys = _run_ffn2(h, W2, b2r, w_row, be, nbt)
    out = _run_sc_combine(ys, gp0, gp1)
    return out.reshape(B, S, D)


# speedup vs baseline: 1.2730x; 1.2730x over previous
import jax
import jax.numpy as jnp
from jax import lax
from jax.experimental import pallas as pl
from jax.experimental.pallas import tpu as pltpu
from jax.experimental.pallas import tpu_sc as plsc

S = 2048
D = 1024
DH = 4096
E = 8
K = 2

C = 128
NB = 40
NPAD = NB * C

DHC = 512
J = DH // DHC

NW = 32
RPW = NPAD // NW
GCH = 5
RPC = RPW // GCH
NBUF = 3
TPW = S // NW
CCH = 2
TPC = TPW // CCH

_SQRT1_2 = 0.7071067811865476


def _gelu(h):
    return 0.5 * h * (1.0 + lax.erf(h * _SQRT1_2))


def _sc_mesh():
    return plsc.VectorSubcoreMesh(core_axis_name="c", subcore_axis_name="s",
                                  num_cores=2, num_subcores=16)


def _gating_kernel(x_ref, gw_ref, gb_ref,
                   gpos0_ref, gpos1_ref, w0_ref, w1_ref,
                   bstart_ref, bend_ref):
    logits = lax.dot_general(
        x_ref[...], gw_ref[...], (((1,), (1,)), ((), ())),
        preferred_element_type=jnp.float32) + gb_ref[...]
    lane = lax.broadcasted_iota(jnp.int32, (S, E), 1)

    m0 = jnp.max(logits, axis=1, keepdims=True)
    i0 = jnp.min(jnp.where(logits == m0, lane, E), axis=1, keepdims=True)
    l1 = jnp.where(lane == i0, -jnp.inf, logits)
    m1 = jnp.max(l1, axis=1, keepdims=True)
    i1 = jnp.min(jnp.where(l1 == m1, lane, E), axis=1, keepdims=True)
    e1 = jnp.exp(m1 - m0)
    w0 = 1.0 / (1.0 + e1)
    w1 = 1.0 - w0

    cnt = (lane == i0).astype(jnp.int32) + (lane == i1).astype(jnp.int32)
    incl = cnt
    k = 1
    while k < S:
        shifted = jnp.concatenate(
            [jnp.zeros((k, E), jnp.int32), incl[:-k]], axis=0)
        incl = incl + shifted
        k *= 2
    excl = incl - cnt

    counts = incl[S - 1:S, :]
    nb = (counts + (C - 1)) // C
    bincl = nb
    k = 1
    while k < E:
        shifted = jnp.concatenate(
            [jnp.zeros((1, k), jnp.int32), bincl[:, :-k]], axis=1)
        bincl = bincl + shifted
        k *= 2
    bstart = bincl - nb
    pstart = C * bstart

    pstart_b = jnp.broadcast_to(pstart, (S, E))
    pos0 = jnp.sum(jnp.where(lane == i0, excl + pstart_b, 0),
                   axis=1, keepdims=True)
    pos1 = jnp.sum(jnp.where(lane == i1, excl + pstart_b, 0),
                   axis=1, keepdims=True)

    gpos0_ref[...] = pos0
    gpos1_ref[...] = pos1
    w0_ref[...] = w0
    w1_ref[...] = w1
    bstart_ref[...] = bstart
    bend_ref[...] = bincl


def _run_gating(x_flat, gate_W, gb):
    return pl.pallas_call(
        _gating_kernel,
        out_shape=(
            jax.ShapeDtypeStruct((S, 1), jnp.int32),
            jax.ShapeDtypeStruct((S, 1), jnp.int32),
            jax.ShapeDtypeStruct((S, 1), jnp.float32),
            jax.ShapeDtypeStruct((S, 1), jnp.float32),
            jax.ShapeDtypeStruct((1, E), jnp.int32),
            jax.ShapeDtypeStruct((1, E), jnp.int32),
        ),
    )(x_flat, gate_W, gb)


def _sc_gather_body(x_hbm, idx_hbm, out_hbm, idx_v, rows_v,
                    gs0, gs1, gs2, ws0, ws1, ws2):
    gsems = [gs0, gs1, gs2]
    wsems = [ws0, ws1, ws2]
    wid = lax.axis_index("s") * 2 + lax.axis_index("c")
    base = wid * RPW
    pltpu.sync_copy(idx_hbm.at[wid], idx_v)
    gets = [None] * GCH
    puts = [None] * GCH
    for c in range(min(NBUF, GCH)):
        gets[c] = pltpu.async_copy(
            x_hbm.at[idx_v.at[c]], rows_v.at[c % NBUF], gsems[c % NBUF])
    for c in range(GCH):
        gets[c].wait()
        puts[c] = pltpu.async_copy(
            rows_v.at[c % NBUF], out_hbm.at[pl.ds(base + c * RPC, RPC)],
            wsems[c % NBUF])
        if c + NBUF < GCH:
            puts[c].wait()
            gets[c + NBUF] = pltpu.async_copy(
                x_hbm.at[idx_v.at[c + NBUF]], rows_v.at[c % NBUF],
                gsems[c % NBUF])
    for c in range(max(0, GCH - NBUF), GCH):
        puts[c].wait()


def _run_sc_gather(x_flat, row_token):
    f = pl.kernel(
        _sc_gather_body,
        out_type=jax.ShapeDtypeStruct((NPAD, D), jnp.float32),
        mesh=_sc_mesh(),
        scratch_types=[
            pltpu.VMEM((GCH, RPC), jnp.int32),
            pltpu.VMEM((NBUF, RPC, D), jnp.float32),
            pltpu.SemaphoreType.DMA,
            pltpu.SemaphoreType.DMA,
            pltpu.SemaphoreType.DMA,
            pltpu.SemaphoreType.DMA,
            pltpu.SemaphoreType.DMA,
            pltpu.SemaphoreType.DMA,
        ],
    )
    return f(x_flat, row_token.reshape(NW, GCH, RPC))


def _ffn_kernel(bstart_ref, bend_ref,
                xs_ref, w1_ref, b1_ref, w2_ref, b2_ref, wr_ref, out_ref):
    e = pl.program_id(0)
    j = pl.program_id(1)
    w1b = w1_ref[0].astype(jnp.bfloat16)
    w2b = w2_ref[0].astype(jnp.bfloat16)
    b1v = b1_ref[0]
    b2v = b2_ref[0]
    lo = bstart_ref[e]
    hi = bend_ref[e]

    def mm1(b):
        xb = xs_ref[pl.ds(b * C, C), :].astype(jnp.bfloat16)
        return lax.dot_general(xb, w1b, (((1,), (0,)), ((), ())),
                               preferred_element_type=jnp.float32)

    def blk(b, hcar):
        hnext = mm1(jnp.minimum(b + 1, NB - 1))
        rs = pl.ds(b * C, C)
        h = _gelu(hcar + b1v)
        wcol = wr_ref[rs, :]
        h = (h * wcol).astype(jnp.bfloat16)
        yb = lax.dot_general(h, w2b, (((1,), (0,)), ((), ())),
                             preferred_element_type=jnp.float32)
        prev = out_ref[rs, :]
        out_ref[rs, :] = jnp.where(j == 0, yb + wcol * b2v, prev + yb)
        return hnext

    lax.fori_loop(lo, hi, blk, mm1(jnp.minimum(lo, NB - 1)))


def _run_ffn(xs, W1, b1r, W2, b2r, w_row, bstart, bend):
    grid_spec = pltpu.PrefetchScalarGridSpec(
        num_scalar_prefetch=2,
        grid=(E, J),
        in_specs=[
            pl.BlockSpec((NPAD, D), lambda e, j, *_: (0, 0)),
            pl.BlockSpec((1, D, DHC), lambda e, j, *_: (e, 0, j)),
            pl.BlockSpec((1, 1, DHC), lambda e, j, *_: (e, 0, j)),
            pl.BlockSpec((1, DHC, D), lambda e, j, *_: (e, j, 0)),
            pl.BlockSpec((1, 1, D), lambda e, j, *_: (e, 0, 0)),
            pl.BlockSpec((NPAD, 1), lambda e, j, *_: (0, 0)),
        ],
        out_specs=pl.BlockSpec((NPAD, D), lambda e, j, *_: (0, 0)),
    )
    return pl.pallas_call(
        _ffn_kernel,
        grid_spec=grid_spec,
        out_shape=jax.ShapeDtypeStruct((NPAD, D), jnp.float32),
    )(bstart, bend, xs, W1, b1r, W2, b2r, w_row)


def _sc_combine_body(ys_hbm, g0_hbm, g1_hbm, out_hbm,
                     g0_v, g1_v, r0_v, r1_v, sem0, sem1):
    wid = lax.axis_index("s") * 2 + lax.axis_index("c")
    base = wid * TPW
    pltpu.sync_copy(g0_hbm.at[wid], g0_v)
    pltpu.sync_copy(g1_hbm.at[wid], g1_v)
    for c in range(CCH):
        cp0 = pltpu.async_copy(ys_hbm.at[g0_v.at[c]], r0_v, sem0)
        cp1 = pltpu.async_copy(ys_hbm.at[g1_v.at[c]], r1_v, sem1)
        cp0.wait()
        cp1.wait()

        def tok(t, carry):
            for v in range(D // 16):
                sl = pl.ds(v * 16, 16)
                r0_v[t, sl] = r0_v[t, sl] + r1_v[t, sl]
            return carry

        lax.fori_loop(0, TPC, tok, 0)
        pltpu.sync_copy(r0_v, out_hbm.at[pl.ds(base + c * TPC, TPC)])


def _run_sc_combine(ys, g0, g1):
    f = pl.kernel(
        _sc_combine_body,
        out_type=jax.ShapeDtypeStruct((S, D), jnp.float32),
        mesh=_sc_mesh(),
        scratch_types=[
            pltpu.VMEM((CCH, TPC), jnp.int32),
            pltpu.VMEM((CCH, TPC), jnp.int32),
            pltpu.VMEM((TPC, D), jnp.float32),
            pltpu.VMEM((TPC, D), jnp.float32),
            pltpu.SemaphoreType.DMA,
            pltpu.SemaphoreType.DMA,
        ],
    )
    return f(ys, g0.reshape(NW, CCH, TPC), g1.reshape(NW, CCH, TPC))


def kernel(x, gate_W, gate_b, W1, b1, W2, b2):
    B = x.shape[0]
    x_flat = x.reshape(S, D)
    gb = gate_b.reshape(1, E)
    b1r = b1.reshape(E, 1, DH)
    b2r = b2.reshape(E, 1, D)

    gpos0, gpos1, w0, w1, bstart, bend = _run_gating(x_flat, gate_W, gb)
    gp0 = gpos0.reshape(S)
    gp1 = gpos1.reshape(S)

    tok_ids = jnp.arange(S, dtype=jnp.int32)
    cat_idx = jnp.concatenate([gp0, gp1])
    row_token = (jnp.zeros((NPAD,), jnp.int32)
                 .at[cat_idx].set(jnp.concatenate([tok_ids, tok_ids])))
    w_row = (jnp.zeros((NPAD,), jnp.float32)
             .at[cat_idx].set(jnp.concatenate([w0.reshape(S),
                                               w1.reshape(S)]))
             ).reshape(NPAD, 1)

    xs = _run_sc_gather(x_flat, row_token)
    ys = _run_ffn(xs, W1, b1r, W2, b2r, w_row,
                  bstart.reshape(E), bend.reshape(E))
    out = _run_sc_combine(ys, gp0, gp1)
    return out.reshape(B, S, D)
